# baseline (speedup 1.0000x reference)
import functools

import jax
import jax.numpy as jnp
from jax import lax
from jax.experimental import pallas as pl
from jax.experimental.pallas import tpu as pltpu, tpu_sc as plsc

N_DST = 10000
N_SRC = 10000
IN_DIM = 128
OUT_DIM = 128
E0 = 160000
E1 = 160000

NC = 2
NS = 16
NW = NC * NS

CH = 128
CPT0 = 104
CPT1 = 56
NCH = NS * (CPT0 + CPT1)
E_PAD = NCH * CH
NPAD = 10112
RPT = NPAD // NS

ROW_BLOCK = 1000


def _proj_body(x0_ref, x1_ref, coeffs_ref, V_ref, h_ref):
    V0 = V_ref[0]
    V1 = V_ref[1]
    W0 = coeffs_ref[0, 0] * V0 + coeffs_ref[0, 1] * V1
    W1 = coeffs_ref[1, 0] * V0 + coeffs_ref[1, 1] * V1
    h_ref[0] = jnp.dot(x0_ref[...], W0, preferred_element_type=jnp.float32)
    h_ref[1] = jnp.dot(x1_ref[...], W1, preferred_element_type=jnp.float32)


def _final_body(acc_ref, xd_ref, Wself_ref, bias_ref, out_ref):
    out_ref[...] = (
        acc_ref[0]
        + acc_ref[1]
        + jnp.dot(xd_ref[...], Wself_ref[...], preferred_element_type=jnp.float32)
        + bias_ref[...]
    )


def _sc_agg_body(h_hbm, zeros_hbm, sidx_hbm, didx_hbm, out_hbm,
                 acc, sidx_v, didx_v, rows_v, sem):
    c = lax.axis_index("c")
    s = lax.axis_index("s")

    pltpu.sync_copy(zeros_hbm, acc.at[pl.ds(s * RPT, RPT)])

    base = jnp.where(c == 0, NS * CPT1 + s * CPT0, s * CPT1)

    @pl.when(c == 0)
    def _():
        pltpu.sync_copy(sidx_hbm.at[pl.ds(base, CPT0)], sidx_v)
        pltpu.sync_copy(didx_hbm.at[pl.ds(base, CPT0)], didx_v)

    @pl.when(c != 0)
    def _():
        pltpu.sync_copy(sidx_hbm.at[pl.ds(base, CPT1)],
                        sidx_v.at[pl.ds(0, CPT1)])
        pltpu.sync_copy(didx_hbm.at[pl.ds(base, CPT1)],
                        didx_v.at[pl.ds(0, CPT1)])

    plsc.subcore_barrier()

    def chunk(j, carry):
        pltpu.async_copy(h_hbm.at[sidx_v.at[j]], rows_v, sem).wait()
        pltpu.sync_copy(rows_v, acc.at[didx_v.at[j]], add=True)
        return carry

    @pl.when(c == 0)
    def _():
        lax.fori_loop(0, CPT0, chunk, 0)

    @pl.when(c != 0)
    def _():
        lax.fori_loop(0, CPT1, chunk, 0)

    plsc.subcore_barrier()

    pltpu.sync_copy(acc.at[pl.ds(s * RPT, RPT)],
                    out_hbm.at[pl.ds(c * NPAD + s * RPT, RPT)])


def _make_sc_agg(interpret=False):
    mesh = plsc.VectorSubcoreMesh(core_axis_name="c", subcore_axis_name="s",
                                  num_cores=NC, num_subcores=NS)
    return pl.kernel(
        _sc_agg_body,
        out_type=jax.ShapeDtypeStruct((NC * NPAD, OUT_DIM), jnp.float32),
        mesh=mesh,
        scratch_types=[
            pltpu.VMEM_SHARED((NPAD, OUT_DIM), jnp.float32),
            pltpu.VMEM((CPT0, CH), jnp.int32),
            pltpu.VMEM((CPT0, CH), jnp.int32),
            pltpu.VMEM((CH, OUT_DIM), jnp.float32),
            pltpu.SemaphoreType.DMA,
        ],
        interpret=interpret,
    )


def kernel(dst_feat, src_feat_0, src_feat_1, edges0_src, edges0_dst,
           edges1_src, edges1_dst, V, coeffs, W_self, bias):
    n_grid = N_SRC // ROW_BLOCK

    h = pl.pallas_call(
        _proj_body,
        grid=(n_grid,),
        in_specs=[
            pl.BlockSpec((ROW_BLOCK, IN_DIM), lambda i: (i, 0)),
            pl.BlockSpec((ROW_BLOCK, IN_DIM), lambda i: (i, 0)),
            pl.BlockSpec(memory_space=pltpu.SMEM),
            pl.BlockSpec((2, IN_DIM, OUT_DIM), lambda i: (0, 0, 0)),
        ],
        out_specs=pl.BlockSpec((2, ROW_BLOCK, OUT_DIM), lambda i: (0, i, 0)),
        out_shape=jax.ShapeDtypeStruct((2, N_SRC, OUT_DIM), jnp.float32),
    )(src_feat_0, src_feat_1, coeffs, V)
    h_flat = h.reshape(2 * N_SRC, OUT_DIM)

    n_pad = E_PAD - (E0 + E1)
    src_all = jnp.concatenate([
        edges0_src, edges1_src + N_SRC,
        jnp.zeros((n_pad,), jnp.int32),
    ]).reshape(NCH, CH)
    dst_all = jnp.concatenate([
        edges0_dst, edges1_dst,
        jnp.full((n_pad,), N_DST, jnp.int32),
    ]).reshape(NCH, CH)

    zeros_blk = jnp.zeros((RPT, OUT_DIM), jnp.float32)

    acc_flat = _make_sc_agg()(h_flat, zeros_blk, src_all, dst_all)
    acc = acc_flat.reshape(NC, NPAD, OUT_DIM)

    bias_2d = bias.reshape(1, OUT_DIM)
    dst_z = pl.pallas_call(
        _final_body,
        grid=(N_DST // ROW_BLOCK,),
        in_specs=[
            pl.BlockSpec((2, ROW_BLOCK, OUT_DIM), lambda i: (0, i, 0)),
            pl.BlockSpec((ROW_BLOCK, IN_DIM), lambda i: (i, 0)),
            pl.BlockSpec((IN_DIM, OUT_DIM), lambda i: (0, 0)),
            pl.BlockSpec((1, OUT_DIM), lambda i: (0, 0)),
        ],
        out_specs=pl.BlockSpec((ROW_BLOCK, OUT_DIM), lambda i: (i, 0)),
        out_shape=jax.ShapeDtypeStruct((N_DST, OUT_DIM), jnp.float32),
    )(acc, dst_feat, W_self, bias_2d)

    att_sc = jnp.ones((2,), dtype=jnp.float32)
    return (dst_z, att_sc)


# speedup vs baseline: 2.8467x; 2.8467x over previous
import functools

import jax
import jax.numpy as jnp
from jax import lax
from jax.experimental import pallas as pl
from jax.experimental.pallas import tpu as pltpu, tpu_sc as plsc

N_DST = 10000
N_SRC = 10000
IN_DIM = 128
OUT_DIM = 128
E0 = 160000
E1 = 160000

NC = 2
NS = 16
NW = NC * NS

CH = 128
CPT = 80
NCH = NW * CPT
E_PAD = NCH * CH
NPAD = 10112
NDUMMY = NPAD - N_DST
RPT = NPAD // NS

ROW_BLOCK = 1000


def _proj_body(x0_ref, x1_ref, coeffs_ref, V_ref, h_ref):
    V0 = V_ref[0]
    V1 = V_ref[1]
    W0 = coeffs_ref[0, 0] * V0 + coeffs_ref[0, 1] * V1
    W1 = coeffs_ref[1, 0] * V0 + coeffs_ref[1, 1] * V1
    h_ref[0] = jnp.dot(x0_ref[...], W0, preferred_element_type=jnp.float32)
    h_ref[1] = jnp.dot(x1_ref[...], W1, preferred_element_type=jnp.float32)


def _final_body(acc_ref, xd_ref, Wself_ref, bias_ref, out_ref):
    out_ref[...] = (
        acc_ref[0]
        + acc_ref[1]
        + jnp.dot(xd_ref[...], Wself_ref[...], preferred_element_type=jnp.float32)
        + bias_ref[...]
    )


def _sc_agg_body(h_hbm, zeros_hbm, sidx_hbm, didx_hbm, out_hbm,
                 acc, sidx_v, didx_v, rows_v, sem):
    c = lax.axis_index("c")
    s = lax.axis_index("s")

    wid = s * NC + c

    pltpu.sync_copy(zeros_hbm, acc.at[pl.ds(s * RPT, RPT)])
    pltpu.sync_copy(sidx_hbm.at[pl.ds(wid * CPT, CPT)], sidx_v)
    pltpu.sync_copy(didx_hbm.at[pl.ds(wid * CPT, CPT)], didx_v)
    plsc.subcore_barrier()

    def chunk(j, carry):
        pltpu.async_copy(h_hbm.at[sidx_v.at[j]], rows_v, sem).wait()
        pltpu.sync_copy(rows_v, acc.at[didx_v.at[j]], add=True)
        return carry

    lax.fori_loop(0, CPT, chunk, 0)
    plsc.subcore_barrier()

    pltpu.sync_copy(acc.at[pl.ds(s * RPT, RPT)],
                    out_hbm.at[pl.ds(c * NPAD + s * RPT, RPT)])


def _make_sc_agg(interpret=False):
    mesh = plsc.VectorSubcoreMesh(core_axis_name="c", subcore_axis_name="s",
                                  num_cores=NC, num_subcores=NS)
    return pl.kernel(
        _sc_agg_body,
        out_type=jax.ShapeDtypeStruct((NC * NPAD, OUT_DIM), jnp.float32),
        mesh=mesh,
        scratch_types=[
            pltpu.VMEM_SHARED((NPAD, OUT_DIM), jnp.float32),
            pltpu.VMEM((CPT, CH), jnp.int32),
            pltpu.VMEM((CPT, CH), jnp.int32),
            pltpu.VMEM((CH, OUT_DIM), jnp.float32),
            pltpu.SemaphoreType.DMA,
        ],
        interpret=interpret,
    )


def kernel(dst_feat, src_feat_0, src_feat_1, edges0_src, edges0_dst,
           edges1_src, edges1_dst, V, coeffs, W_self, bias):
    n_grid = N_SRC // ROW_BLOCK

    h = pl.pallas_call(
        _proj_body,
        grid=(n_grid,),
        in_specs=[
            pl.BlockSpec((ROW_BLOCK, IN_DIM), lambda i: (i, 0)),
            pl.BlockSpec((ROW_BLOCK, IN_DIM), lambda i: (i, 0)),
            pl.BlockSpec(memory_space=pltpu.SMEM),
            pl.BlockSpec((2, IN_DIM, OUT_DIM), lambda i: (0, 0, 0)),
        ],
        out_specs=pl.BlockSpec((2, ROW_BLOCK, OUT_DIM), lambda i: (0, i, 0)),
        out_shape=jax.ShapeDtypeStruct((2, N_SRC, OUT_DIM), jnp.float32),
    )(src_feat_0, src_feat_1, coeffs, V)
    h_flat = h.reshape(2 * N_SRC, OUT_DIM)

    n_pad = E_PAD - (E0 + E1)
    pad_iota = jnp.arange(n_pad, dtype=jnp.int32)
    src_all = jnp.concatenate([
        edges0_src, edges1_src + N_SRC,
        pad_iota % (2 * N_SRC),
    ]).reshape(NCH, CH)
    dst_all = jnp.concatenate([
        edges0_dst, edges1_dst,
        N_DST + pad_iota % NDUMMY,
    ]).reshape(NCH, CH)

    zeros_blk = jnp.zeros((RPT, OUT_DIM), jnp.float32)

    acc_flat = _make_sc_agg()(h_flat, zeros_blk, src_all, dst_all)
    acc = acc_flat.reshape(NC, NPAD, OUT_DIM)

    bias_2d = bias.reshape(1, OUT_DIM)
    dst_z = pl.pallas_call(
        _final_body,
        grid=(N_DST // ROW_BLOCK,),
        in_specs=[
            pl.BlockSpec((2, ROW_BLOCK, OUT_DIM), lambda i: (0, i, 0)),
            pl.BlockSpec((ROW_BLOCK, IN_DIM), lambda i: (i, 0)),
            pl.BlockSpec((IN_DIM, OUT_DIM), lambda i: (0, 0)),
            pl.BlockSpec((1, OUT_DIM), lambda i: (0, 0)),
        ],
        out_specs=pl.BlockSpec((ROW_BLOCK, OUT_DIM), lambda i: (i, 0)),
        out_shape=jax.ShapeDtypeStruct((N_DST, OUT_DIM), jnp.float32),
    )(acc, dst_feat, W_self, bias_2d)

    att_sc = jnp.ones((2,), dtype=jnp.float32)
    return (dst_z, att_sc)


# speedup vs baseline: 4.0515x; 1.4232x over previous
import functools

import jax
import jax.numpy as jnp
from jax import lax
from jax.experimental import pallas as pl
from jax.experimental.pallas import tpu as pltpu, tpu_sc as plsc

N_DST = 10000
N_SRC = 10000
IN_DIM = 128
OUT_DIM = 128
E0 = 160000
E1 = 160000

NC = 2
NS = 16
NW = NC * NS

CH = 128
CPT = 80
H_CPT = CPT // 2
H_NPAIR = H_CPT // 2
NCH = NW * CPT
E_PAD = NCH * CH
NPAD = 10112
NDUMMY = NPAD - N_DST
RPT = NPAD // NS

ROW_BLOCK = 1000


def _proj_body(x0_ref, x1_ref, coeffs_ref, V_ref, h_ref):
    V0 = V_ref[0]
    V1 = V_ref[1]
    W0 = coeffs_ref[0, 0] * V0 + coeffs_ref[0, 1] * V1
    W1 = coeffs_ref[1, 0] * V0 + coeffs_ref[1, 1] * V1
    h_ref[0] = jnp.dot(x0_ref[...], W0, preferred_element_type=jnp.float32)
    h_ref[1] = jnp.dot(x1_ref[...], W1, preferred_element_type=jnp.float32)


def _final_body(acc_ref, xd_ref, Wself_ref, bias_ref, out_ref):
    out_ref[...] = (
        acc_ref[0]
        + acc_ref[1]
        + jnp.dot(xd_ref[...], Wself_ref[...], preferred_element_type=jnp.float32)
        + bias_ref[...]
    )


def _sc_agg_body(h_hbm, zeros_hbm, sidx_hbm, didx_hbm, out_hbm,
                 acc, sidx_v, didx_v, rows0, rows1, sem0, sem1):
    c = lax.axis_index("c")
    s = lax.axis_index("s")

    wid = s * NC + c

    pltpu.sync_copy(zeros_hbm, acc.at[pl.ds(s * RPT, RPT)])
    plsc.subcore_barrier()

    for half in range(2):
        base = wid * CPT + half * H_CPT
        pltpu.sync_copy(sidx_hbm.at[pl.ds(base, H_CPT)], sidx_v)
        pltpu.sync_copy(didx_hbm.at[pl.ds(base, H_CPT)], didx_v)
        pltpu.async_copy(h_hbm.at[sidx_v.at[0]], rows0, sem0)

        def body(i, carry):
            j0 = 2 * i
            j1 = j0 + 1
            pltpu.async_copy(h_hbm.at[sidx_v.at[j1]], rows1, sem1)
            pltpu.make_async_copy(h_hbm.at[sidx_v.at[j0]], rows0, sem0).wait()
            pltpu.sync_copy(rows0, acc.at[didx_v.at[j0]], add=True)

            @pl.when(i + 1 < H_NPAIR)
            def _():
                pltpu.async_copy(h_hbm.at[sidx_v.at[j0 + 2]], rows0, sem0)

            pltpu.make_async_copy(h_hbm.at[sidx_v.at[j1]], rows1, sem1).wait()
            pltpu.sync_copy(rows1, acc.at[didx_v.at[j1]], add=True)
            return carry

        lax.fori_loop(0, H_NPAIR, body, 0)
    plsc.subcore_barrier()

    pltpu.sync_copy(acc.at[pl.ds(s * RPT, RPT)],
                    out_hbm.at[pl.ds(c * NPAD + s * RPT, RPT)])


def _make_sc_agg(interpret=False):
    mesh = plsc.VectorSubcoreMesh(core_axis_name="c", subcore_axis_name="s",
                                  num_cores=NC, num_subcores=NS)
    return pl.kernel(
        _sc_agg_body,
        out_type=jax.ShapeDtypeStruct((NC * NPAD, OUT_DIM), jnp.float32),
        mesh=mesh,
        scratch_types=[
            pltpu.VMEM_SHARED((NPAD, OUT_DIM), jnp.float32),
            pltpu.VMEM((H_CPT, CH), jnp.int32),
            pltpu.VMEM((H_CPT, CH), jnp.int32),
            pltpu.VMEM((CH, OUT_DIM), jnp.float32),
            pltpu.VMEM((CH, OUT_DIM), jnp.float32),
            pltpu.SemaphoreType.DMA,
            pltpu.SemaphoreType.DMA,
        ],
        interpret=interpret,
    )


def kernel(dst_feat, src_feat_0, src_feat_1, edges0_src, edges0_dst,
           edges1_src, edges1_dst, V, coeffs, W_self, bias):
    n_grid = N_SRC // ROW_BLOCK

    h = pl.pallas_call(
        _proj_body,
        grid=(n_grid,),
        in_specs=[
            pl.BlockSpec((ROW_BLOCK, IN_DIM), lambda i: (i, 0)),
            pl.BlockSpec((ROW_BLOCK, IN_DIM), lambda i: (i, 0)),
            pl.BlockSpec(memory_space=pltpu.SMEM),
            pl.BlockSpec((2, IN_DIM, OUT_DIM), lambda i: (0, 0, 0)),
        ],
        out_specs=pl.BlockSpec((2, ROW_BLOCK, OUT_DIM), lambda i: (0, i, 0)),
        out_shape=jax.ShapeDtypeStruct((2, N_SRC, OUT_DIM), jnp.float32),
    )(src_feat_0, src_feat_1, coeffs, V)
    h_flat = h.reshape(2 * N_SRC, OUT_DIM)

    n_pad = E_PAD - (E0 + E1)
    pad_iota = jnp.arange(n_pad, dtype=jnp.int32)
    src_all = jnp.concatenate([
        edges0_src, edges1_src + N_SRC,
        pad_iota % (2 * N_SRC),
    ]).reshape(NCH, CH)
    dst_all = jnp.concatenate([
        edges0_dst, edges1_dst,
        N_DST + pad_iota % NDUMMY,
    ]).reshape(NCH, CH)

    zeros_blk = jnp.zeros((RPT, OUT_DIM), jnp.float32)

    acc_flat = _make_sc_agg()(h_flat, zeros_blk, src_all, dst_all)
    acc = acc_flat.reshape(NC, NPAD, OUT_DIM)

    bias_2d = bias.reshape(1, OUT_DIM)
    dst_z = pl.pallas_call(
        _final_body,
        grid=(N_DST // ROW_BLOCK,),
        in_specs=[
            pl.BlockSpec((2, ROW_BLOCK, OUT_DIM), lambda i: (0, i, 0)),
            pl.BlockSpec((ROW_BLOCK, IN_DIM), lambda i: (i, 0)),
            pl.BlockSpec((IN_DIM, OUT_DIM), lambda i: (0, 0)),
            pl.BlockSpec((1, OUT_DIM), lambda i: (0, 0)),
        ],
        out_specs=pl.BlockSpec((ROW_BLOCK, OUT_DIM), lambda i: (i, 0)),
        out_shape=jax.ShapeDtypeStruct((N_DST, OUT_DIM), jnp.float32),
    )(acc, dst_feat, W_self, bias_2d)

    att_sc = jnp.ones((2,), dtype=jnp.float32)
    return (dst_z, att_sc)


# speedup vs baseline: 4.1478x; 1.0238x over previous
import functools

import jax
import jax.numpy as jnp
from jax import lax
from jax.experimental import pallas as pl
from jax.experimental.pallas import tpu as pltpu, tpu_sc as plsc

N_DST = 10000
N_SRC = 10000
IN_DIM = 128
OUT_DIM = 128
E0 = 160000
E1 = 160000

NC = 2
NS = 16
NW = NC * NS

CH = 128
CPT = 80
H_CPT = CPT // 2
H_NPAIR = H_CPT // 2
NCH = NW * CPT
E_PAD = NCH * CH
NPAD = 10112
NDUMMY = NPAD - N_DST
RPT = NPAD // NS

ROW_BLOCK = 1000


def _proj_body(x0_ref, x1_ref, coeffs_ref, V_ref, h_ref):
    V0 = V_ref[0]
    V1 = V_ref[1]
    W0 = coeffs_ref[0, 0] * V0 + coeffs_ref[0, 1] * V1
    W1 = coeffs_ref[1, 0] * V0 + coeffs_ref[1, 1] * V1
    h_ref[0] = jnp.dot(x0_ref[...], W0, preferred_element_type=jnp.float32)
    h_ref[1] = jnp.dot(x1_ref[...], W1, preferred_element_type=jnp.float32)


def _final_body(acc_ref, xd_ref, Wself_ref, bias_ref, out_ref):
    out_ref[...] = (
        acc_ref[0]
        + acc_ref[1]
        + jnp.dot(xd_ref[...], Wself_ref[...], preferred_element_type=jnp.float32)
        + bias_ref[...]
    )


def _sc_agg_body(h_hbm, sidx_hbm, didx_hbm, out_hbm,
                 acc, sidx_v, didx_v, rows0, rows1, sem0, sem1):
    c = lax.axis_index("c")
    s = lax.axis_index("s")

    wid = s * NC + c

    zv = jnp.zeros((16,), jnp.float32)

    def zbody(i, carry):
        for k in range(OUT_DIM // 16):
            rows0[i, pl.ds(16 * k, 16)] = zv
        return carry

    lax.fori_loop(0, CH, zbody, 0)
    for k in range(RPT // CH):
        pltpu.sync_copy(rows0, acc.at[pl.ds(s * RPT + k * CH, CH)])
    rem = RPT % CH
    if rem:
        pltpu.sync_copy(rows0.at[pl.ds(0, rem)],
                        acc.at[pl.ds(s * RPT + (RPT // CH) * CH, rem)])
    plsc.subcore_barrier()

    for half in range(2):
        base = wid * CPT + half * H_CPT
        pltpu.sync_copy(sidx_hbm.at[pl.ds(base, H_CPT)], sidx_v)
        pltpu.sync_copy(didx_hbm.at[pl.ds(base, H_CPT)], didx_v)
        pltpu.async_copy(h_hbm.at[sidx_v.at[0]], rows0, sem0)

        def body(i, carry):
            j0 = 2 * i
            j1 = j0 + 1
            pltpu.async_copy(h_hbm.at[sidx_v.at[j1]], rows1, sem1)
            pltpu.make_async_copy(h_hbm.at[sidx_v.at[j0]], rows0, sem0).wait()
            pltpu.sync_copy(rows0, acc.at[didx_v.at[j0]], add=True)

            @pl.when(i + 1 < H_NPAIR)
            def _():
                pltpu.async_copy(h_hbm.at[sidx_v.at[j0 + 2]], rows0, sem0)

            pltpu.make_async_copy(h_hbm.at[sidx_v.at[j1]], rows1, sem1).wait()
            pltpu.sync_copy(rows1, acc.at[didx_v.at[j1]], add=True)
            return carry

        lax.fori_loop(0, H_NPAIR, body, 0)
    plsc.subcore_barrier()

    pltpu.sync_copy(acc.at[pl.ds(s * RPT, RPT)],
                    out_hbm.at[pl.ds(c * NPAD + s * RPT, RPT)])


def _make_sc_agg(interpret=False):
    mesh = plsc.VectorSubcoreMesh(core_axis_name="c", subcore_axis_name="s",
                                  num_cores=NC, num_subcores=NS)
    return pl.kernel(
        _sc_agg_body,
        out_type=jax.ShapeDtypeStruct((NC * NPAD, OUT_DIM), jnp.float32),
        mesh=mesh,
        scratch_types=[
            pltpu.VMEM_SHARED((NPAD, OUT_DIM), jnp.float32),
            pltpu.VMEM((H_CPT, CH), jnp.int32),
            pltpu.VMEM((H_CPT, CH), jnp.int32),
            pltpu.VMEM((CH, OUT_DIM), jnp.float32),
            pltpu.VMEM((CH, OUT_DIM), jnp.float32),
            pltpu.SemaphoreType.DMA,
            pltpu.SemaphoreType.DMA,
        ],
        interpret=interpret,
    )


def kernel(dst_feat, src_feat_0, src_feat_1, edges0_src, edges0_dst,
           edges1_src, edges1_dst, V, coeffs, W_self, bias):
    n_grid = N_SRC // ROW_BLOCK

    h = pl.pallas_call(
        _proj_body,
        grid=(n_grid,),
        in_specs=[
            pl.BlockSpec((ROW_BLOCK, IN_DIM), lambda i: (i, 0)),
            pl.BlockSpec((ROW_BLOCK, IN_DIM), lambda i: (i, 0)),
            pl.BlockSpec(memory_space=pltpu.SMEM),
            pl.BlockSpec((2, IN_DIM, OUT_DIM), lambda i: (0, 0, 0)),
        ],
        out_specs=pl.BlockSpec((2, ROW_BLOCK, OUT_DIM), lambda i: (0, i, 0)),
        out_shape=jax.ShapeDtypeStruct((2, N_SRC, OUT_DIM), jnp.float32),
    )(src_feat_0, src_feat_1, coeffs, V)
    h_flat = h.reshape(2 * N_SRC, OUT_DIM)

    n_pad = E_PAD - (E0 + E1)
    pad_iota = jnp.arange(n_pad, dtype=jnp.int32)
    src_all = jnp.concatenate([
        edges0_src, edges1_src + N_SRC,
        pad_iota % (2 * N_SRC),
    ]).reshape(NCH, CH)
    dst_all = jnp.concatenate([
        edges0_dst, edges1_dst,
        N_DST + pad_iota % NDUMMY,
    ]).reshape(NCH, CH)

    acc_flat = _make_sc_agg()(h_flat, src_all, dst_all)
    acc = acc_flat.reshape(NC, NPAD, OUT_DIM)

    bias_2d = bias.reshape(1, OUT_DIM)
    dst_z = pl.pallas_call(
        _final_body,
        grid=(N_DST // ROW_BLOCK,),
        in_specs=[
            pl.BlockSpec((2, ROW_BLOCK, OUT_DIM), lambda i: (0, i, 0)),
            pl.BlockSpec((ROW_BLOCK, IN_DIM), lambda i: (i, 0)),
            pl.BlockSpec((IN_DIM, OUT_DIM), lambda i: (0, 0)),
            pl.BlockSpec((1, OUT_DIM), lambda i: (0, 0)),
        ],
        out_specs=pl.BlockSpec((ROW_BLOCK, OUT_DIM), lambda i: (i, 0)),
        out_shape=jax.ShapeDtypeStruct((N_DST, OUT_DIM), jnp.float32),
    )(acc, dst_feat, W_self, bias_2d)

    att_sc = jnp.ones((2,), dtype=jnp.float32)
    return (dst_z, att_sc)


# speedup vs baseline: 4.3297x; 1.0439x over previous
import functools

import jax
import jax.numpy as jnp
from jax import lax
from jax.experimental import pallas as pl
from jax.experimental.pallas import tpu as pltpu, tpu_sc as plsc

N_DST = 10000
N_SRC = 10000
IN_DIM = 128
OUT_DIM = 128
E0 = 160000
E1 = 160000

NC = 2
NS = 16
NW = NC * NS

CH = 128
CPT = 80
H_CPT = CPT // 2
H_NPAIR = H_CPT // 2
NCH = NW * CPT
E_PAD = NCH * CH
NPAD = 10112
NDUMMY = NPAD - N_DST
RPT = NPAD // NS

ROW_BLOCK = 2000


def _proj_body(x0_ref, x1_ref, coeffs_ref, V_ref, h_ref):
    V0 = V_ref[0]
    V1 = V_ref[1]
    W0 = coeffs_ref[0, 0] * V0 + coeffs_ref[0, 1] * V1
    W1 = coeffs_ref[1, 0] * V0 + coeffs_ref[1, 1] * V1
    h_ref[0] = jnp.dot(x0_ref[...], W0, preferred_element_type=jnp.float32)
    h_ref[1] = jnp.dot(x1_ref[...], W1, preferred_element_type=jnp.float32)


def _final_body(acc_ref, xd_ref, Wself_ref, bias_ref, out_ref):
    out_ref[...] = (
        acc_ref[0]
        + acc_ref[1]
        + jnp.dot(xd_ref[...], Wself_ref[...], preferred_element_type=jnp.float32)
        + bias_ref[...]
    )


def _sc_agg_body(h_hbm, sidx_hbm, didx_hbm, out_hbm,
                 acc, sidx_v, didx_v, rows0, rows1, sem0, sem1):
    c = lax.axis_index("c")
    s = lax.axis_index("s")

    wid = s * NC + c

    zv = jnp.zeros((16,), jnp.float32)

    def zbody(i, carry):
        for k in range(OUT_DIM // 16):
            rows0[i, pl.ds(16 * k, 16)] = zv
        return carry

    lax.fori_loop(0, CH, zbody, 0)
    for k in range(RPT // CH):
        pltpu.sync_copy(rows0, acc.at[pl.ds(s * RPT + k * CH, CH)])
    rem = RPT % CH
    if rem:
        pltpu.sync_copy(rows0.at[pl.ds(0, rem)],
                        acc.at[pl.ds(s * RPT + (RPT // CH) * CH, rem)])
    plsc.subcore_barrier()

    for half in range(2):
        base = wid * CPT + half * H_CPT
        pltpu.sync_copy(sidx_hbm.at[pl.ds(base, H_CPT)], sidx_v)
        pltpu.sync_copy(didx_hbm.at[pl.ds(base, H_CPT)], didx_v)
        pltpu.async_copy(h_hbm.at[sidx_v.at[0]], rows0, sem0)

        def body(i, carry):
            j0 = 2 * i
            j1 = j0 + 1
            pltpu.async_copy(h_hbm.at[sidx_v.at[j1]], rows1, sem1)
            pltpu.make_async_copy(h_hbm.at[sidx_v.at[j0]], rows0, sem0).wait()
            pltpu.sync_copy(rows0, acc.at[didx_v.at[j0]], add=True)

            @pl.when(i + 1 < H_NPAIR)
            def _():
                pltpu.async_copy(h_hbm.at[sidx_v.at[j0 + 2]], rows0, sem0)

            pltpu.make_async_copy(h_hbm.at[sidx_v.at[j1]], rows1, sem1).wait()
            pltpu.sync_copy(rows1, acc.at[didx_v.at[j1]], add=True)
            return carry

        lax.fori_loop(0, H_NPAIR, body, 0)
    plsc.subcore_barrier()

    pltpu.sync_copy(acc.at[pl.ds(s * RPT, RPT)],
                    out_hbm.at[pl.ds(c * NPAD + s * RPT, RPT)])


def _make_sc_agg(interpret=False):
    mesh = plsc.VectorSubcoreMesh(core_axis_name="c", subcore_axis_name="s",
                                  num_cores=NC, num_subcores=NS)
    return pl.kernel(
        _sc_agg_body,
        out_type=jax.ShapeDtypeStruct((NC * NPAD, OUT_DIM), jnp.float32),
        mesh=mesh,
        scratch_types=[
            pltpu.VMEM_SHARED((NPAD, OUT_DIM), jnp.float32),
            pltpu.VMEM((H_CPT, CH), jnp.int32),
            pltpu.VMEM((H_CPT, CH), jnp.int32),
            pltpu.VMEM((CH, OUT_DIM), jnp.float32),
            pltpu.VMEM((CH, OUT_DIM), jnp.float32),
            pltpu.SemaphoreType.DMA,
            pltpu.SemaphoreType.DMA,
        ],
        interpret=interpret,
    )


def kernel(dst_feat, src_feat_0, src_feat_1, edges0_src, edges0_dst,
           edges1_src, edges1_dst, V, coeffs, W_self, bias):
    n_grid = N_SRC // ROW_BLOCK

    h = pl.pallas_call(
        _proj_body,
        grid=(n_grid,),
        in_specs=[
            pl.BlockSpec((ROW_BLOCK, IN_DIM), lambda i: (i, 0)),
            pl.BlockSpec((ROW_BLOCK, IN_DIM), lambda i: (i, 0)),
            pl.BlockSpec(memory_space=pltpu.SMEM),
            pl.BlockSpec((2, IN_DIM, OUT_DIM), lambda i: (0, 0, 0)),
        ],
        out_specs=pl.BlockSpec((2, ROW_BLOCK, OUT_DIM), lambda i: (0, i, 0)),
        out_shape=jax.ShapeDtypeStruct((2, N_SRC, OUT_DIM), jnp.float32),
    )(src_feat_0, src_feat_1, coeffs, V)
    h_flat = h.reshape(2 * N_SRC, OUT_DIM)

    n_pad = E_PAD - (E0 + E1)
    pad_iota = jnp.arange(n_pad, dtype=jnp.int32)
    src_all = jnp.concatenate([
        edges0_src, edges1_src + N_SRC,
        pad_iota % (2 * N_SRC),
    ]).reshape(NCH, CH)
    dst_all = jnp.concatenate([
        edges0_dst, edges1_dst,
        N_DST + pad_iota % NDUMMY,
    ]).reshape(NCH, CH)

    acc_flat = _make_sc_agg()(h_flat, src_all, dst_all)
    acc = acc_flat.reshape(NC, NPAD, OUT_DIM)

    bias_2d = bias.reshape(1, OUT_DIM)
    dst_z = pl.pallas_call(
        _final_body,
        grid=(N_DST // ROW_BLOCK,),
        in_specs=[
            pl.BlockSpec((2, ROW_BLOCK, OUT_DIM), lambda i: (0, i, 0)),
            pl.BlockSpec((ROW_BLOCK, IN_DIM), lambda i: (i, 0)),
            pl.BlockSpec((IN_DIM, OUT_DIM), lambda i: (0, 0)),
            pl.BlockSpec((1, OUT_DIM), lambda i: (0, 0)),
        ],
        out_specs=pl.BlockSpec((ROW_BLOCK, OUT_DIM), lambda i: (i, 0)),
        out_shape=jax.ShapeDtypeStruct((N_DST, OUT_DIM), jnp.float32),
    )(acc, dst_feat, W_self, bias_2d)

    att_sc = jnp.ones((2,), dtype=jnp.float32)
    return (dst_z, att_sc)


# speedup vs baseline: 4.4080x; 1.0181x over previous
import functools

import jax
import jax.numpy as jnp
from jax import lax
from jax.experimental import pallas as pl
from jax.experimental.pallas import tpu as pltpu, tpu_sc as plsc

N_DST = 10000
N_SRC = 10000
IN_DIM = 128
OUT_DIM = 128
E0 = 160000
E1 = 160000

NC = 2
NS = 16
NW = NC * NS

CH = 128
CPT = 80
H_CPT = CPT // 2
H_NPAIR = H_CPT // 2
NCH = NW * CPT
E_PAD = NCH * CH
NPAD = 10112
NDUMMY = NPAD - N_DST
RPT = NPAD // NS

ROW_BLOCK = 5000


def _proj_body(x0_ref, x1_ref, coeffs_ref, V_ref, h_ref):
    V0 = V_ref[0]
    V1 = V_ref[1]
    W0 = coeffs_ref[0, 0] * V0 + coeffs_ref[0, 1] * V1
    W1 = coeffs_ref[1, 0] * V0 + coeffs_ref[1, 1] * V1
    h_ref[0] = jnp.dot(x0_ref[...], W0, preferred_element_type=jnp.float32)
    h_ref[1] = jnp.dot(x1_ref[...], W1, preferred_element_type=jnp.float32)


def _final_body(acc_ref, xd_ref, Wself_ref, bias_ref, out_ref):
    out_ref[...] = (
        acc_ref[0]
        + acc_ref[1]
        + jnp.dot(xd_ref[...], Wself_ref[...], preferred_element_type=jnp.float32)
        + bias_ref[...]
    )


def _sc_agg_body(h_hbm, sidx_hbm, didx_hbm, out_hbm,
                 acc, sidx_v, didx_v, rows0, rows1, sem0, sem1):
    c = lax.axis_index("c")
    s = lax.axis_index("s")

    wid = s * NC + c

    zv = jnp.zeros((16,), jnp.float32)

    def zbody(i, carry):
        for k in range(OUT_DIM // 16):
            rows0[i, pl.ds(16 * k, 16)] = zv
        return carry

    lax.fori_loop(0, CH, zbody, 0)
    for k in range(RPT // CH):
        pltpu.sync_copy(rows0, acc.at[pl.ds(s * RPT + k * CH, CH)])
    rem = RPT % CH
    if rem:
        pltpu.sync_copy(rows0.at[pl.ds(0, rem)],
                        acc.at[pl.ds(s * RPT + (RPT // CH) * CH, rem)])
    plsc.subcore_barrier()

    for half in range(2):
        base = wid * CPT + half * H_CPT
        pltpu.sync_copy(sidx_hbm.at[pl.ds(base, H_CPT)], sidx_v)
        pltpu.sync_copy(didx_hbm.at[pl.ds(base, H_CPT)], didx_v)
        pltpu.async_copy(h_hbm.at[sidx_v.at[0]], rows0, sem0)

        def body(i, carry):
            j0 = 2 * i
            j1 = j0 + 1
            pltpu.async_copy(h_hbm.at[sidx_v.at[j1]], rows1, sem1)
            pltpu.make_async_copy(h_hbm.at[sidx_v.at[j0]], rows0, sem0).wait()
            pltpu.sync_copy(rows0, acc.at[didx_v.at[j0]], add=True)

            @pl.when(i + 1 < H_NPAIR)
            def _():
                pltpu.async_copy(h_hbm.at[sidx_v.at[j0 + 2]], rows0, sem0)

            pltpu.make_async_copy(h_hbm.at[sidx_v.at[j1]], rows1, sem1).wait()
            pltpu.sync_copy(rows1, acc.at[didx_v.at[j1]], add=True)
            return carry

        lax.fori_loop(0, H_NPAIR, body, 0)
    plsc.subcore_barrier()

    pltpu.sync_copy(acc.at[pl.ds(s * RPT, RPT)],
                    out_hbm.at[pl.ds(c * NPAD + s * RPT, RPT)])


def _make_sc_agg(interpret=False):
    mesh = plsc.VectorSubcoreMesh(core_axis_name="c", subcore_axis_name="s",
                                  num_cores=NC, num_subcores=NS)
    return pl.kernel(
        _sc_agg_body,
        out_type=jax.ShapeDtypeStruct((NC * NPAD, OUT_DIM), jnp.float32),
        mesh=mesh,
        scratch_types=[
            pltpu.VMEM_SHARED((NPAD, OUT_DIM), jnp.float32),
            pltpu.VMEM((H_CPT, CH), jnp.int32),
            pltpu.VMEM((H_CPT, CH), jnp.int32),
            pltpu.VMEM((CH, OUT_DIM), jnp.float32),
            pltpu.VMEM((CH, OUT_DIM), jnp.float32),
            pltpu.SemaphoreType.DMA,
            pltpu.SemaphoreType.DMA,
        ],
        interpret=interpret,
    )


def kernel(dst_feat, src_feat_0, src_feat_1, edges0_src, edges0_dst,
           edges1_src, edges1_dst, V, coeffs, W_self, bias):
    n_grid = N_SRC // ROW_BLOCK

    h = pl.pallas_call(
        _proj_body,
        grid=(n_grid,),
        in_specs=[
            pl.BlockSpec((ROW_BLOCK, IN_DIM), lambda i: (i, 0)),
            pl.BlockSpec((ROW_BLOCK, IN_DIM), lambda i: (i, 0)),
            pl.BlockSpec(memory_space=pltpu.SMEM),
            pl.BlockSpec((2, IN_DIM, OUT_DIM), lambda i: (0, 0, 0)),
        ],
        out_specs=pl.BlockSpec((2, ROW_BLOCK, OUT_DIM), lambda i: (0, i, 0)),
        out_shape=jax.ShapeDtypeStruct((2, N_SRC, OUT_DIM), jnp.float32),
    )(src_feat_0, src_feat_1, coeffs, V)
    h_flat = h.reshape(2 * N_SRC, OUT_DIM)

    n_pad = E_PAD - (E0 + E1)
    pad_iota = jnp.arange(n_pad, dtype=jnp.int32)
    src_all = jnp.concatenate([
        edges0_src, edges1_src + N_SRC,
        pad_iota % (2 * N_SRC),
    ]).reshape(NCH, CH)
    dst_all = jnp.concatenate([
        edges0_dst, edges1_dst,
        N_DST + pad_iota % NDUMMY,
    ]).reshape(NCH, CH)

    acc_flat = _make_sc_agg()(h_flat, src_all, dst_all)
    acc = acc_flat.reshape(NC, NPAD, OUT_DIM)

    bias_2d = bias.reshape(1, OUT_DIM)
    dst_z = pl.pallas_call(
        _final_body,
        grid=(N_DST // ROW_BLOCK,),
        in_specs=[
            pl.BlockSpec((2, ROW_BLOCK, OUT_DIM), lambda i: (0, i, 0)),
            pl.BlockSpec((ROW_BLOCK, IN_DIM), lambda i: (i, 0)),
            pl.BlockSpec((IN_DIM, OUT_DIM), lambda i: (0, 0)),
            pl.BlockSpec((1, OUT_DIM), lambda i: (0, 0)),
        ],
        out_specs=pl.BlockSpec((ROW_BLOCK, OUT_DIM), lambda i: (i, 0)),
        out_shape=jax.ShapeDtypeStruct((N_DST, OUT_DIM), jnp.float32),
    )(acc, dst_feat, W_self, bias_2d)

    att_sc = jnp.ones((2,), dtype=jnp.float32)
    return (dst_z, att_sc)


# speedup vs baseline: 4.4101x; 1.0005x over previous
import jax
import jax.numpy as jnp
from jax import lax
from jax.experimental import pallas as pl
from jax.experimental.pallas import tpu as pltpu, tpu_sc as plsc

N_DST = 10000
N_SRC = 10000
IN_DIM = 128
OUT_DIM = 128
E0 = 160000
E1 = 160000

NC = 2
NS = 16
NW = NC * NS

CH = 128
CPT = 80
H_CPT = CPT // 2
H_NPAIR = H_CPT // 2
NCH = NW * CPT
E_PAD = NCH * CH
NPAD = 10112
NDUMMY = NPAD - N_DST
RPT = NPAD // NS

ROW_BLOCK = 5000


def _proj_body(x0_ref, x1_ref, coeffs_ref, V_ref, h_ref):
    V0 = V_ref[0]
    V1 = V_ref[1]
    W0 = coeffs_ref[0, 0] * V0 + coeffs_ref[0, 1] * V1
    W1 = coeffs_ref[1, 0] * V0 + coeffs_ref[1, 1] * V1
    h_ref[0] = jnp.dot(x0_ref[...], W0, preferred_element_type=jnp.float32)
    h_ref[1] = jnp.dot(x1_ref[...], W1, preferred_element_type=jnp.float32)


def _final_body(acc_ref, xd_ref, Wself_ref, bias_ref, out_ref):
    out_ref[...] = (
        acc_ref[0]
        + acc_ref[1]
        + jnp.dot(xd_ref[...], Wself_ref[...], preferred_element_type=jnp.float32)
        + bias_ref[...]
    )


def _sc_agg_body(h_hbm, sidx_hbm, didx_hbm, out_hbm,
                 acc, sidx_v, didx_v, rows0, rows1, sem0, sem1):
    c = lax.axis_index("c")
    s = lax.axis_index("s")

    wid = s * NC + c

    zv = jnp.zeros((16,), jnp.float32)

    def zbody(i, carry):
        for k in range(OUT_DIM // 16):
            rows0[i, pl.ds(16 * k, 16)] = zv
        return carry

    lax.fori_loop(0, CH, zbody, 0)
    for k in range(RPT // CH):
        pltpu.sync_copy(rows0, acc.at[pl.ds(s * RPT + k * CH, CH)])
    rem = RPT % CH
    if rem:
        pltpu.sync_copy(rows0.at[pl.ds(0, rem)],
                        acc.at[pl.ds(s * RPT + (RPT // CH) * CH, rem)])
    plsc.subcore_barrier()

    for half in range(2):
        base = wid * CPT + half * H_CPT
        pltpu.sync_copy(sidx_hbm.at[pl.ds(base, H_CPT)], sidx_v)
        pltpu.sync_copy(didx_hbm.at[pl.ds(base, H_CPT)], didx_v)
        pltpu.async_copy(h_hbm.at[sidx_v.at[0]], rows0, sem0)

        def body(i, carry):
            j0 = 2 * i
            j1 = j0 + 1
            pltpu.async_copy(h_hbm.at[sidx_v.at[j1]], rows1, sem1)
            pltpu.make_async_copy(h_hbm.at[sidx_v.at[j0]], rows0, sem0).wait()
            pltpu.sync_copy(rows0, acc.at[didx_v.at[j0]], add=True)

            @pl.when(i + 1 < H_NPAIR)
            def _():
                pltpu.async_copy(h_hbm.at[sidx_v.at[j0 + 2]], rows0, sem0)

            pltpu.make_async_copy(h_hbm.at[sidx_v.at[j1]], rows1, sem1).wait()
            pltpu.sync_copy(rows1, acc.at[didx_v.at[j1]], add=True)
            return carry

        lax.fori_loop(0, H_NPAIR, body, 0)
    plsc.subcore_barrier()

    pltpu.sync_copy(acc.at[pl.ds(s * RPT, RPT)],
                    out_hbm.at[pl.ds(c * NPAD + s * RPT, RPT)])


def _make_sc_agg():
    mesh = plsc.VectorSubcoreMesh(core_axis_name="c", subcore_axis_name="s",
                                  num_cores=NC, num_subcores=NS)
    return pl.kernel(
        _sc_agg_body,
        out_type=jax.ShapeDtypeStruct((NC * NPAD, OUT_DIM), jnp.float32),
        mesh=mesh,
        scratch_types=[
            pltpu.VMEM_SHARED((NPAD, OUT_DIM), jnp.float32),
            pltpu.VMEM((H_CPT, CH), jnp.int32),
            pltpu.VMEM((H_CPT, CH), jnp.int32),
            pltpu.VMEM((CH, OUT_DIM), jnp.float32),
            pltpu.VMEM((CH, OUT_DIM), jnp.float32),
            pltpu.SemaphoreType.DMA,
            pltpu.SemaphoreType.DMA,
        ],
    )


def kernel(dst_feat, src_feat_0, src_feat_1, edges0_src, edges0_dst,
           edges1_src, edges1_dst, V, coeffs, W_self, bias):
    n_grid = N_SRC // ROW_BLOCK

    h = pl.pallas_call(
        _proj_body,
        grid=(n_grid,),
        in_specs=[
            pl.BlockSpec((ROW_BLOCK, IN_DIM), lambda i: (i, 0)),
            pl.BlockSpec((ROW_BLOCK, IN_DIM), lambda i: (i, 0)),
            pl.BlockSpec(memory_space=pltpu.SMEM),
            pl.BlockSpec((2, IN_DIM, OUT_DIM), lambda i: (0, 0, 0)),
        ],
        out_specs=pl.BlockSpec((2, ROW_BLOCK, OUT_DIM), lambda i: (0, i, 0)),
        out_shape=jax.ShapeDtypeStruct((2, N_SRC, OUT_DIM), jnp.float32),
    )(src_feat_0, src_feat_1, coeffs, V)
    h_flat = h.reshape(2 * N_SRC, OUT_DIM)

    n_pad = E_PAD - (E0 + E1)
    pad_iota = jnp.arange(n_pad, dtype=jnp.int32)
    src_all = jnp.concatenate([
        edges0_src, edges1_src + N_SRC,
        pad_iota % (2 * N_SRC),
    ]).reshape(NCH, CH)
    dst_all = jnp.concatenate([
        edges0_dst, edges1_dst,
        N_DST + pad_iota % NDUMMY,
    ]).reshape(NCH, CH)

    acc_flat = _make_sc_agg()(h_flat, src_all, dst_all)
    acc = acc_flat.reshape(NC, NPAD, OUT_DIM)

    bias_2d = bias.reshape(1, OUT_DIM)
    dst_z = pl.pallas_call(
        _final_body,
        grid=(N_DST // ROW_BLOCK,),
        in_specs=[
            pl.BlockSpec((2, ROW_BLOCK, OUT_DIM), lambda i: (0, i, 0)),
            pl.BlockSpec((ROW_BLOCK, IN_DIM), lambda i: (i, 0)),
            pl.BlockSpec((IN_DIM, OUT_DIM), lambda i: (0, 0)),
            pl.BlockSpec((1, OUT_DIM), lambda i: (0, 0)),
        ],
        out_specs=pl.BlockSpec((ROW_BLOCK, OUT_DIM), lambda i: (i, 0)),
        out_shape=jax.ShapeDtypeStruct((N_DST, OUT_DIM), jnp.float32),
    )(acc, dst_feat, W_self, bias_2d)

    att_sc = jnp.ones((2,), dtype=jnp.float32)
    return (dst_z, att_sc)
